# baseline (speedup 1.0000x reference)
import functools

import jax
import jax.numpy as jnp
from jax import lax
from jax.experimental import pallas as pl
from jax.experimental.pallas import tpu as pltpu
from jax.experimental.pallas import tpu_sc as plsc

NC = 2
NS = 16
NW = NC * NS
CHUNK = 128


def _sc_aggregate(x, src_p, dst_p, zrows, *, n_pad, rows_per_sub, n_chunks):
    d = x.shape[1]
    per_w = n_chunks * CHUNK
    mesh = plsc.VectorSubcoreMesh(core_axis_name="c", subcore_axis_name="s")

    @functools.partial(
        pl.kernel,
        out_type=jax.ShapeDtypeStruct((NC, n_pad, d), jnp.float32),
        mesh=mesh,
        scratch_types=[
            pltpu.VMEM_SHARED((n_pad, d), jnp.float32),
            pltpu.VMEM((CHUNK,), jnp.int32),
            pltpu.VMEM((CHUNK,), jnp.int32),
            pltpu.VMEM((CHUNK,), jnp.int32),
            pltpu.VMEM((CHUNK,), jnp.int32),
            pltpu.VMEM((CHUNK, d), jnp.float32),
            pltpu.VMEM((CHUNK, d), jnp.float32),
            pltpu.SemaphoreType.DMA,
            pltpu.SemaphoreType.DMA,
        ],
    )
    def agg(x_hbm, src_hbm, dst_hbm, z_hbm, out_hbm, acc_sh,
            sidx_a, didx_a, sidx_b, didx_b, rows_a, rows_b, gsem_a, gsem_b):
        cid = lax.axis_index("c")
        sid = lax.axis_index("s")
        wid = sid * NC + cid
        r0 = sid * rows_per_sub
        e0 = wid * per_w

        def idx_sync(j, sidx, didx):
            base = pl.multiple_of(e0 + j * CHUNK, 8)
            pltpu.sync_copy(src_hbm.at[pl.ds(base, CHUNK)], sidx)
            pltpu.sync_copy(dst_hbm.at[pl.ds(base, CHUNK)], didx)

        def scatter_sync(didx, rows):
            pltpu.sync_copy(rows, acc_sh.at[didx], add=True)

        pltpu.sync_copy(z_hbm, acc_sh.at[pl.ds(r0, rows_per_sub)])
        plsc.subcore_barrier()

        idx_sync(0, sidx_a, didx_a)
        idx_sync(1, sidx_b, didx_b)

        def body(t, carry):
            j = 2 * t
            ga = pltpu.async_copy(x_hbm.at[sidx_a], rows_a, gsem_a)
            gb = pltpu.async_copy(x_hbm.at[sidx_b], rows_b, gsem_b)
            ga.wait()
            scatter_sync(didx_a, rows_a)
            gb.wait()
            scatter_sync(didx_b, rows_b)
            idx_sync(j + 2, sidx_a, didx_a)
            idx_sync(j + 3, sidx_b, didx_b)
            return carry

        lax.fori_loop(0, n_chunks // 2, body, 0)

        plsc.subcore_barrier()
        pltpu.sync_copy(acc_sh.at[pl.ds(r0, rows_per_sub)],
                        out_hbm.at[cid, pl.ds(r0, rows_per_sub)])

    return agg(x, src_p, dst_p, zrows)


def _tc_body(p0_ref, p1_ref, x_ref, wn_ref, wr_ref, bn_ref, br_ref, o_ref):
    neigh = p0_ref[...] + p1_ref[...]
    o_ref[...] = (
        jnp.dot(neigh, wn_ref[...], preferred_element_type=jnp.float32)
        + jnp.dot(x_ref[...], wr_ref[...], preferred_element_type=jnp.float32)
        + bn_ref[...] + br_ref[...]
    )


def _tc_dense(p0, p1, x, wn, wr, bn, br):
    m, d = x.shape
    bm = 1000
    dn = wn.shape[1]
    return pl.pallas_call(
        _tc_body,
        grid=(m // bm,),
        in_specs=[
            pl.BlockSpec((bm, d), lambda i: (i, 0)),
            pl.BlockSpec((bm, d), lambda i: (i, 0)),
            pl.BlockSpec((bm, d), lambda i: (i, 0)),
            pl.BlockSpec((d, dn), lambda i: (0, 0)),
            pl.BlockSpec((d, dn), lambda i: (0, 0)),
            pl.BlockSpec((1, dn), lambda i: (0, 0)),
            pl.BlockSpec((1, dn), lambda i: (0, 0)),
        ],
        out_specs=pl.BlockSpec((bm, dn), lambda i: (i, 0)),
        out_shape=jax.ShapeDtypeStruct((m, dn), jnp.float32),
    )(p0, p1, x, wn, wr, bn.reshape(1, dn), br.reshape(1, dn))


def kernel(x, edge_index, W_neigh, b_neigh, W_root, b_root):
    n, d = x.shape
    e = edge_index.shape[1]
    src = edge_index[0].astype(jnp.int32)
    dst = edge_index[1].astype(jnp.int32)

    rows_per_sub = -(-(n + 1) // (NS * 8)) * 8
    n_pad = NS * rows_per_sub

    per_w = -(-e // NW)
    n_chunks = -(-(-(-per_w // CHUNK)) // 8) * 8
    e_pad = (NW * n_chunks + 2) * CHUNK
    pad = e_pad - e
    src_p = jnp.concatenate([src, jnp.zeros((pad,), jnp.int32)])
    dst_p = jnp.concatenate(
        [dst, n + (jnp.arange(pad, dtype=jnp.int32) % (n_pad - n))])
    zrows = jnp.zeros((rows_per_sub, d), jnp.float32)

    partial = _sc_aggregate(x, src_p, dst_p, zrows,
                            n_pad=n_pad, rows_per_sub=rows_per_sub,
                            n_chunks=n_chunks)
    return _tc_dense(partial[0, :n], partial[1, :n], x,
                     W_neigh, W_root, b_neigh, b_root)


# speedup vs baseline: 1.5786x; 1.5786x over previous
import functools

import jax
import jax.numpy as jnp
from jax import lax
from jax.experimental import pallas as pl
from jax.experimental.pallas import tpu as pltpu
from jax.experimental.pallas import tpu_sc as plsc

NC = 2
NS = 16
CHUNK = 128


def _sc_aggregate(xb, src_p, dst_p, zrows, *, n_pad, rows_per_sub, n_chunks):
    dh = xb.shape[2]
    per_s = n_chunks * CHUNK
    mesh = plsc.VectorSubcoreMesh(core_axis_name="c", subcore_axis_name="s")

    @functools.partial(
        pl.kernel,
        out_type=jax.ShapeDtypeStruct((NC, n_pad, dh), jnp.float32),
        mesh=mesh,
        scratch_types=[
            pltpu.VMEM_SHARED((n_pad, dh), jnp.float32),
            pltpu.VMEM_SHARED((n_pad, dh), jnp.float32),
            pltpu.VMEM((CHUNK,), jnp.int32),
            pltpu.VMEM((CHUNK,), jnp.int32),
            pltpu.VMEM((CHUNK, dh), jnp.float32),
            pltpu.SemaphoreType.DMA,
        ],
    )
    def agg(xb_hbm, src_hbm, dst_hbm, z_hbm, out_hbm,
            x_sh, acc_sh, sidx, didx, rows, gsem):
        cid = lax.axis_index("c")
        sid = lax.axis_index("s")
        r0 = sid * rows_per_sub
        e0 = sid * per_s

        pltpu.sync_copy(xb_hbm.at[cid, pl.ds(r0, rows_per_sub)],
                        x_sh.at[pl.ds(r0, rows_per_sub)])
        pltpu.sync_copy(z_hbm, acc_sh.at[pl.ds(r0, rows_per_sub)])
        plsc.subcore_barrier()

        def body(j, carry):
            base = pl.multiple_of(e0 + j * CHUNK, 8)
            pltpu.sync_copy(src_hbm.at[pl.ds(base, CHUNK)], sidx)
            pltpu.sync_copy(dst_hbm.at[pl.ds(base, CHUNK)], didx)
            pltpu.async_copy(x_sh.at[sidx], rows, gsem).wait()
            pltpu.sync_copy(rows, acc_sh.at[didx], add=True)
            return carry

        lax.fori_loop(0, n_chunks, body, 0)
        plsc.subcore_barrier()
        pltpu.sync_copy(acc_sh.at[pl.ds(r0, rows_per_sub)],
                        out_hbm.at[cid, pl.ds(r0, rows_per_sub)])

    return agg(xb, src_p, dst_p, zrows)


def _tc_body(p0_ref, p1_ref, x_ref, wn0_ref, wn1_ref, wr_ref, bn_ref, br_ref,
             o_ref):
    o_ref[...] = (
        jnp.dot(p0_ref[...], wn0_ref[...], preferred_element_type=jnp.float32)
        + jnp.dot(p1_ref[...], wn1_ref[...],
                  preferred_element_type=jnp.float32)
        + jnp.dot(x_ref[...], wr_ref[...], preferred_element_type=jnp.float32)
        + bn_ref[...] + br_ref[...]
    )


def _tc_dense(p0, p1, x, wn0, wn1, wr, bn, br):
    m, d = x.shape
    bm = 1000
    dh = wn0.shape[0]
    dn = wr.shape[1]
    return pl.pallas_call(
        _tc_body,
        grid=(m // bm,),
        in_specs=[
            pl.BlockSpec((bm, dh), lambda i: (i, 0)),
            pl.BlockSpec((bm, dh), lambda i: (i, 0)),
            pl.BlockSpec((bm, d), lambda i: (i, 0)),
            pl.BlockSpec((dh, dn), lambda i: (0, 0)),
            pl.BlockSpec((dh, dn), lambda i: (0, 0)),
            pl.BlockSpec((d, dn), lambda i: (0, 0)),
            pl.BlockSpec((1, dn), lambda i: (0, 0)),
            pl.BlockSpec((1, dn), lambda i: (0, 0)),
        ],
        out_specs=pl.BlockSpec((bm, dn), lambda i: (i, 0)),
        out_shape=jax.ShapeDtypeStruct((m, dn), jnp.float32),
    )(p0, p1, x, wn0, wn1, wr, bn.reshape(1, dn), br.reshape(1, dn))


def kernel(x, edge_index, W_neigh, b_neigh, W_root, b_root):
    n, d = x.shape
    dh = d // NC
    e = edge_index.shape[1]
    src = edge_index[0].astype(jnp.int32)
    dst = edge_index[1].astype(jnp.int32)

    rows_per_sub = -(-(n + 1) // (NS * 8)) * 8
    n_pad = NS * rows_per_sub

    per_s = -(-e // NS)
    n_chunks = -(-(-(-per_s // CHUNK)) // 8) * 8
    e_pad = NS * n_chunks * CHUNK
    pad = e_pad - e
    src_p = jnp.concatenate([src, jnp.zeros((pad,), jnp.int32)])
    dst_p = jnp.concatenate(
        [dst, n + (jnp.arange(pad, dtype=jnp.int32) % (n_pad - n))])

    xb = jnp.pad(x, ((0, n_pad - n), (0, 0))).reshape(n_pad, NC, dh)
    xb = jnp.moveaxis(xb, 1, 0)
    zrows = jnp.zeros((rows_per_sub, dh), jnp.float32)

    partial = _sc_aggregate(xb, src_p, dst_p, zrows,
                            n_pad=n_pad, rows_per_sub=rows_per_sub,
                            n_chunks=n_chunks)
    return _tc_dense(partial[0, :n], partial[1, :n], x,
                     W_neigh[:dh], W_neigh[dh:], W_root, b_neigh, b_root)
